# baseline (speedup 1.0000x reference)
import jax
import jax.numpy as jnp
from jax.experimental import pallas as pl
from jax.experimental.pallas import tpu as pltpu

T_BLOCK = 4


def _add_kernel(x_ref, pos_ref, out_ref):
    i = pl.program_id(0)
    pos = pos_ref[pl.ds(i * T_BLOCK, T_BLOCK), :]
    out_ref[...] = x_ref[...] + pos[:, :, None]


def kernel(x, pos_table):
    batch, maxlen, dim = x.shape
    xt = jnp.transpose(x, (1, 2, 0))
    grid = (maxlen // T_BLOCK,)
    out = pl.pallas_call(
        _add_kernel,
        grid=grid,
        in_specs=[
            pl.BlockSpec((T_BLOCK, dim, batch), lambda i: (i, 0, 0)),
            pl.BlockSpec((maxlen, dim), lambda i: (0, 0)),
        ],
        out_specs=pl.BlockSpec((T_BLOCK, dim, batch), lambda i: (i, 0, 0)),
        out_shape=jax.ShapeDtypeStruct((maxlen, dim, batch), x.dtype),
        compiler_params=pltpu.CompilerParams(
            dimension_semantics=("parallel",)),
    )(xt, pos_table)
    return out.transpose(2, 0, 1)


# speedup vs baseline: 1.0150x; 1.0150x over previous
import jax
import jax.numpy as jnp
from jax.experimental import pallas as pl
from jax.experimental.pallas import tpu as pltpu

T_BLOCK = 8


def _add_kernel(x_ref, pos_ref, out_ref):
    i = pl.program_id(0)
    pos = pos_ref[pl.ds(i * T_BLOCK, T_BLOCK), :]
    out_ref[...] = x_ref[...] + pos[:, :, None]


def kernel(x, pos_table):
    batch, maxlen, dim = x.shape
    xt = jnp.transpose(x, (1, 2, 0))
    grid = (maxlen // T_BLOCK,)
    out = pl.pallas_call(
        _add_kernel,
        grid=grid,
        in_specs=[
            pl.BlockSpec((T_BLOCK, dim, batch), lambda i: (i, 0, 0)),
            pl.BlockSpec((maxlen, dim), lambda i: (0, 0)),
        ],
        out_specs=pl.BlockSpec((T_BLOCK, dim, batch), lambda i: (i, 0, 0)),
        out_shape=jax.ShapeDtypeStruct((maxlen, dim, batch), x.dtype),
        compiler_params=pltpu.CompilerParams(
            dimension_semantics=("parallel",)),
    )(xt, pos_table)
    return out.transpose(2, 0, 1)


# speedup vs baseline: 1.0174x; 1.0024x over previous
import jax
import jax.numpy as jnp
from jax.experimental import pallas as pl
from jax.experimental.pallas import tpu as pltpu

T_BLOCK = 10


def _add_kernel(x_ref, pos_ref, out_ref):
    i = pl.program_id(0)
    pos = pos_ref[pl.ds(i * T_BLOCK, T_BLOCK), :]
    out_ref[...] = x_ref[...] + pos[:, :, None]


def kernel(x, pos_table):
    batch, maxlen, dim = x.shape
    xt = jnp.transpose(x, (1, 2, 0))
    grid = (maxlen // T_BLOCK,)
    out = pl.pallas_call(
        _add_kernel,
        grid=grid,
        in_specs=[
            pl.BlockSpec((T_BLOCK, dim, batch), lambda i: (i, 0, 0)),
            pl.BlockSpec((maxlen, dim), lambda i: (0, 0)),
        ],
        out_specs=pl.BlockSpec((T_BLOCK, dim, batch), lambda i: (i, 0, 0)),
        out_shape=jax.ShapeDtypeStruct((maxlen, dim, batch), x.dtype),
        compiler_params=pltpu.CompilerParams(
            dimension_semantics=("parallel",)),
    )(xt, pos_table)
    return out.transpose(2, 0, 1)
